# baseline (speedup 1.0000x reference)
import functools

import jax
import jax.numpy as jnp
from jax import lax
from jax.experimental import pallas as pl
from jax.experimental.pallas import tpu as pltpu

H = 1024
E = 8
I = 1408
S = 2816
N = 2048
BT = 256


def _router_body(x_ref, gw_ref, w_ref):
    x = x_ref[...]
    logits = lax.dot_general(x, gw_ref[...], (((1,), (1,)), ((), ())),
                             preferred_element_type=jnp.float32)
    mx = jnp.max(logits, axis=-1, keepdims=True)
    ex = jnp.exp(logits - mx)
    scores = ex / jnp.sum(ex, axis=-1, keepdims=True)
    iota = lax.broadcasted_iota(jnp.int32, scores.shape, 1)
    m1 = jnp.max(scores, axis=-1, keepdims=True)
    i1 = jnp.min(jnp.where(scores == m1, iota, E), axis=-1, keepdims=True)
    sel1 = iota == i1
    masked = jnp.where(sel1, -jnp.inf, scores)
    m2 = jnp.max(masked, axis=-1, keepdims=True)
    i2 = jnp.min(jnp.where(masked == m2, iota, E), axis=-1, keepdims=True)
    sel2 = iota == i2
    w_ref[...] = jnp.where(sel1 | sel2, scores, 0.0)


def _expert_body(x_ref, w_ref, gu_ref, dn_ref, out_ref):
    e = pl.program_id(0)
    t = pl.program_id(1)
    xb = x_ref[...]
    gu = lax.dot_general(xb, gu_ref[0], (((1,), (1,)), ((), ())),
                         preferred_element_type=jnp.float32)
    g = gu[:, :I]
    u = gu[:, I:]
    h = (g * jax.nn.sigmoid(g) * u).astype(jnp.bfloat16)
    eo = lax.dot_general(h, dn_ref[0], (((1,), (1,)), ((), ())),
                         preferred_element_type=jnp.float32)
    wb = w_ref[...]
    iota = lax.broadcasted_iota(jnp.int32, wb.shape, 1)
    wcol = jnp.sum(jnp.where(iota == e, wb, 0.0), axis=1, keepdims=True)
    contrib = wcol * eo
    base = pl.ds(t * BT, BT)

    @pl.when(e == 0)
    def _():
        out_ref[base, :] = contrib

    @pl.when(e != 0)
    def _():
        out_ref[base, :] += contrib


def _shared_body(acc_ref, x_ref, sg_ref, su_ref, sd_ref, seg_ref, out_ref):
    xb = x_ref[...]
    g = lax.dot_general(xb, sg_ref[...], (((1,), (1,)), ((), ())),
                        preferred_element_type=jnp.float32)
    u = lax.dot_general(xb, su_ref[...], (((1,), (1,)), ((), ())),
                        preferred_element_type=jnp.float32)
    h = (g * jax.nn.sigmoid(g) * u).astype(jnp.bfloat16)
    sh = lax.dot_general(h, sd_ref[...], (((1,), (1,)), ((), ())),
                         preferred_element_type=jnp.float32)
    gate_logit = jnp.sum(xb.astype(jnp.float32) * seg_ref[...],
                         axis=1, keepdims=True)
    gate = jax.nn.sigmoid(gate_logit)
    out_ref[...] = acc_ref[...] + gate * sh


def kernel(x, gate_w, experts_gate_up, experts_down, shared_gate_w,
           shared_up_w, shared_down_w, shared_expert_gate_w):
    Bx, Tx, Hx = x.shape
    xf = x.reshape(Bx * Tx, Hx)
    xbf = xf.astype(jnp.bfloat16)
    gu_bf = experts_gate_up.astype(jnp.bfloat16)
    dn_bf = experts_down.astype(jnp.bfloat16)
    sg_bf = shared_gate_w.astype(jnp.bfloat16)
    su_bf = shared_up_w.astype(jnp.bfloat16)
    sd_bf = shared_down_w.astype(jnp.bfloat16)

    w = pl.pallas_call(
        _router_body,
        grid=(N // BT,),
        in_specs=[
            pl.BlockSpec((BT, H), lambda t: (t, 0)),
            pl.BlockSpec((E, H), lambda t: (0, 0)),
        ],
        out_specs=pl.BlockSpec((BT, E), lambda t: (t, 0)),
        out_shape=jax.ShapeDtypeStruct((N, E), jnp.float32),
    )(xf, gate_w)

    moe_out = pl.pallas_call(
        _expert_body,
        grid=(E, N // BT),
        in_specs=[
            pl.BlockSpec((BT, H), lambda e, t: (t, 0)),
            pl.BlockSpec((BT, E), lambda e, t: (t, 0)),
            pl.BlockSpec((1, 2 * I, H), lambda e, t: (e, 0, 0)),
            pl.BlockSpec((1, H, I), lambda e, t: (e, 0, 0)),
        ],
        out_specs=pl.BlockSpec((N, H), lambda e, t: (0, 0)),
        out_shape=jax.ShapeDtypeStruct((N, H), jnp.float32),
    )(xbf, w, gu_bf, dn_bf)

    out = pl.pallas_call(
        _shared_body,
        grid=(N // BT,),
        in_specs=[
            pl.BlockSpec((BT, H), lambda t: (t, 0)),
            pl.BlockSpec((BT, H), lambda t: (t, 0)),
            pl.BlockSpec((S, H), lambda t: (0, 0)),
            pl.BlockSpec((S, H), lambda t: (0, 0)),
            pl.BlockSpec((H, S), lambda t: (0, 0)),
            pl.BlockSpec((1, H), lambda t: (0, 0)),
        ],
        out_specs=pl.BlockSpec((BT, H), lambda t: (t, 0)),
        out_shape=jax.ShapeDtypeStruct((N, H), jnp.float32),
        input_output_aliases={0: 0},
    )(moe_out, xbf, sg_bf, su_bf, sd_bf, shared_expert_gate_w)

    return out.reshape(Bx, Tx, Hx)
